# baseline (speedup 1.0000x reference)
import jax
import jax.numpy as jnp
from jax.experimental import pallas as pl


def _fp1_body(x_ref, w1_ref, o_ref):
    o_ref[...] = jnp.dot(x_ref[...], w1_ref[...],
                         preferred_element_type=jnp.float32)


def _layer1_body(adj_ref, fp1_ref, b1_ref, o_ref):
    u = jnp.dot(adj_ref[...], fp1_ref[...],
                preferred_element_type=jnp.float32)
    o_ref[...] = jnp.maximum(u + b1_ref[...], 0.0)


def _layer2_body(adj_ref, h_ref, w2_ref, b2_ref, u2_ref, res_ref):
    t = jnp.dot(adj_ref[...], h_ref[...],
                preferred_element_type=jnp.float32)
    u2 = jnp.dot(t, w2_ref[...],
                 preferred_element_type=jnp.float32) + b2_ref[...]
    u2_ref[...] = u2
    m = jnp.max(u2, axis=1, keepdims=True)
    lse = jnp.log(jnp.sum(jnp.exp(u2 - m), axis=1, keepdims=True)) + m
    res_ref[...] = u2 - lse


def _pick_bm(m):
    for bm in (400, 200, 100, 50, 25, 8):
        if m % bm == 0:
            return bm
    return m


def kernel(x, adj, W1, b1, W2, b2):
    m, nfeat = x.shape
    nhid = W1.shape[1]
    ncls = W2.shape[1]
    bm = _pick_bm(m)

    fp1 = pl.pallas_call(
        _fp1_body,
        out_shape=jax.ShapeDtypeStruct((m, nhid), jnp.float32),
    )(x, W1)

    h = pl.pallas_call(
        _layer1_body,
        grid=(m // bm,),
        in_specs=[
            pl.BlockSpec((bm, m), lambda i: (i, 0)),
            pl.BlockSpec((m, nhid), lambda i: (0, 0)),
            pl.BlockSpec((1, nhid), lambda i: (0, 0)),
        ],
        out_specs=pl.BlockSpec((bm, nhid), lambda i: (i, 0)),
        out_shape=jax.ShapeDtypeStruct((m, nhid), jnp.float32),
    )(adj, fp1, b1.reshape(1, nhid))

    u2, res = pl.pallas_call(
        _layer2_body,
        grid=(m // bm,),
        in_specs=[
            pl.BlockSpec((bm, m), lambda i: (i, 0)),
            pl.BlockSpec((m, nhid), lambda i: (0, 0)),
            pl.BlockSpec((nhid, ncls), lambda i: (0, 0)),
            pl.BlockSpec((1, ncls), lambda i: (0, 0)),
        ],
        out_specs=[
            pl.BlockSpec((bm, ncls), lambda i: (i, 0)),
            pl.BlockSpec((bm, ncls), lambda i: (i, 0)),
        ],
        out_shape=[
            jax.ShapeDtypeStruct((m, ncls), jnp.float32),
            jax.ShapeDtypeStruct((m, ncls), jnp.float32),
        ],
    )(adj, h, W2, b2.reshape(1, ncls))

    return (res, fp1, u2)


# speedup vs baseline: 1.0104x; 1.0104x over previous
import jax
import jax.numpy as jnp
from jax.experimental import pallas as pl


def _fp1_body(x_ref, w1_ref, o_ref):
    o_ref[...] = jnp.dot(x_ref[...], w1_ref[...],
                         preferred_element_type=jnp.float32)


def _layer1_body(adj_ref, fp1_ref, b1_ref, o_ref):
    u = jnp.dot(adj_ref[...].astype(jnp.bfloat16),
                fp1_ref[...].astype(jnp.bfloat16),
                preferred_element_type=jnp.float32)
    o_ref[...] = jnp.maximum(u + b1_ref[...], 0.0)


def _layer2_body(adj_ref, h_ref, w2_ref, b2_ref, u2_ref, res_ref):
    t = jnp.dot(adj_ref[...].astype(jnp.bfloat16),
                h_ref[...].astype(jnp.bfloat16),
                preferred_element_type=jnp.float32)
    u2 = jnp.dot(t, w2_ref[...],
                 preferred_element_type=jnp.float32) + b2_ref[...]
    u2_ref[...] = u2
    m = jnp.max(u2, axis=1, keepdims=True)
    lse = jnp.log(jnp.sum(jnp.exp(u2 - m), axis=1, keepdims=True)) + m
    res_ref[...] = u2 - lse


def _pick_bm(m):
    for bm in (400, 200, 100, 50, 25, 8):
        if m % bm == 0:
            return bm
    return m


def kernel(x, adj, W1, b1, W2, b2):
    m, nfeat = x.shape
    nhid = W1.shape[1]
    ncls = W2.shape[1]
    bm = _pick_bm(m)

    fp1 = pl.pallas_call(
        _fp1_body,
        out_shape=jax.ShapeDtypeStruct((m, nhid), jnp.float32),
    )(x, W1)

    h = pl.pallas_call(
        _layer1_body,
        grid=(m // bm,),
        in_specs=[
            pl.BlockSpec((bm, m), lambda i: (i, 0)),
            pl.BlockSpec((m, nhid), lambda i: (0, 0)),
            pl.BlockSpec((1, nhid), lambda i: (0, 0)),
        ],
        out_specs=pl.BlockSpec((bm, nhid), lambda i: (i, 0)),
        out_shape=jax.ShapeDtypeStruct((m, nhid), jnp.float32),
    )(adj, fp1, b1.reshape(1, nhid))

    u2, res = pl.pallas_call(
        _layer2_body,
        grid=(m // bm,),
        in_specs=[
            pl.BlockSpec((bm, m), lambda i: (i, 0)),
            pl.BlockSpec((m, nhid), lambda i: (0, 0)),
            pl.BlockSpec((nhid, ncls), lambda i: (0, 0)),
            pl.BlockSpec((1, ncls), lambda i: (0, 0)),
        ],
        out_specs=[
            pl.BlockSpec((bm, ncls), lambda i: (i, 0)),
            pl.BlockSpec((bm, ncls), lambda i: (i, 0)),
        ],
        out_shape=[
            jax.ShapeDtypeStruct((m, ncls), jnp.float32),
            jax.ShapeDtypeStruct((m, ncls), jnp.float32),
        ],
    )(adj, h, W2, b2.reshape(1, ncls))

    return (res, fp1, u2)


# speedup vs baseline: 1.0438x; 1.0330x over previous
import functools

import jax
import jax.numpy as jnp
from jax.experimental import pallas as pl
from jax.experimental.pallas import tpu as pltpu


def _body(bm, x_ref, adj_ref, w1_ref, b1_ref, w2_ref, b2_ref,
          fp1_ref, u2_ref, res_ref, fp1b_ref, h_ref):
    p = pl.program_id(0)
    i = pl.program_id(1)

    @pl.when((p == 0) & (i == 0))
    def _():
        fp1 = jnp.dot(x_ref[...], w1_ref[...],
                      preferred_element_type=jnp.float32)
        fp1_ref[...] = fp1
        fp1b_ref[...] = fp1.astype(jnp.bfloat16)

    a16 = adj_ref[...].astype(jnp.bfloat16)

    @pl.when(p == 0)
    def _():
        u = jnp.dot(a16, fp1b_ref[...], preferred_element_type=jnp.float32)
        h_ref[pl.ds(i * bm, bm), :] = jnp.maximum(
            u + b1_ref[...], 0.0).astype(jnp.bfloat16)

    @pl.when(p == 1)
    def _():
        t = jnp.dot(a16, h_ref[...], preferred_element_type=jnp.float32)
        u2 = jnp.dot(t, w2_ref[...],
                     preferred_element_type=jnp.float32) + b2_ref[...]
        u2_ref[...] = u2
        mx = jnp.max(u2, axis=1, keepdims=True)
        lse = jnp.log(jnp.sum(jnp.exp(u2 - mx), axis=1, keepdims=True)) + mx
        res_ref[...] = u2 - lse


def _pick_bm(m):
    for bm in (400, 200, 100, 50, 25, 8):
        if m % bm == 0:
            return bm
    return m


def kernel(x, adj, W1, b1, W2, b2):
    m, nfeat = x.shape
    nhid = W1.shape[1]
    ncls = W2.shape[1]
    bm = _pick_bm(m)

    fp1, u2, res = pl.pallas_call(
        functools.partial(_body, bm),
        grid=(2, m // bm),
        in_specs=[
            pl.BlockSpec((m, nfeat), lambda p, i: (0, 0)),
            pl.BlockSpec((bm, m), lambda p, i: (i, 0)),
            pl.BlockSpec((nfeat, nhid), lambda p, i: (0, 0)),
            pl.BlockSpec((1, nhid), lambda p, i: (0, 0)),
            pl.BlockSpec((nhid, ncls), lambda p, i: (0, 0)),
            pl.BlockSpec((1, ncls), lambda p, i: (0, 0)),
        ],
        out_specs=[
            pl.BlockSpec((m, nhid), lambda p, i: (0, 0)),
            pl.BlockSpec((bm, ncls), lambda p, i: (i * p, 0)),
            pl.BlockSpec((bm, ncls), lambda p, i: (i * p, 0)),
        ],
        out_shape=[
            jax.ShapeDtypeStruct((m, nhid), jnp.float32),
            jax.ShapeDtypeStruct((m, ncls), jnp.float32),
            jax.ShapeDtypeStruct((m, ncls), jnp.float32),
        ],
        scratch_shapes=[
            pltpu.VMEM((m, nhid), jnp.bfloat16),
            pltpu.VMEM((m, nhid), jnp.bfloat16),
        ],
    )(x, adj, W1, b1.reshape(1, nhid), W2, b2.reshape(1, ncls))

    return (res, fp1, u2)
